# Initial kernel scaffold; baseline (speedup 1.0000x reference)
import jax
import jax.numpy as jnp
from jax.experimental import pallas as pl


def kernel(x_tab, x_gnn, edge_index, edge_attr, params):
    raise NotImplementedError("write your pallas kernel here")


# speedup vs baseline: 1.0005x; 1.0005x over previous
import functools
import jax
import jax.numpy as jnp
import numpy as np
from jax.experimental import pallas as pl
from jax.experimental.pallas import tpu as pltpu

C = 128; NHEAD = 8; H = 128; S = 12; N = 10000; FUSED = C + 2 * H
AVG_LOG = float(np.log(17.0))


def _ln(x, g, b, eps=1e-5):
    m = jnp.mean(x, axis=-1, keepdims=True)
    v = jnp.mean((x - m) ** 2, axis=-1, keepdims=True)
    return (x - m) / jnp.sqrt(v + eps) * g + b


def _fuse_mlp_kernel(x_ref, lng_ref, lnb_ref, w1_ref, b1_ref, w2_ref, b2_ref,
                     w3_ref, b3_ref, ng_ref, nb_ref, o_ref):
    x = x_ref[...]
    hh = _ln(x, lng_ref[...], lnb_ref[...])
    hh = hh @ w1_ref[...].T + b1_ref[...]
    hh = jnp.where(hh > 0, hh, 0.01 * hh)
    hh = hh @ w2_ref[...].T + b2_ref[...]
    hh = jnp.where(hh > 0, hh, 0.01 * hh)
    hh = hh @ w3_ref[...].T + b3_ref[...]
    o_ref[...] = (x + _ln(hh, ng_ref[...], nb_ref[...])) * 0.5


def _fuse_mlp(x, fp):
    Bq = x.shape[0]
    blk = 512
    grid = Bq // blk
    wspec = pl.BlockSpec((FUSED, FUSED), lambda i: (0, 0))
    vspec = pl.BlockSpec((FUSED,), lambda i: (0,))
    return pl.pallas_call(
        _fuse_mlp_kernel,
        grid=(grid,),
        in_specs=[pl.BlockSpec((blk, FUSED), lambda i: (i, 0)),
                  vspec, vspec, wspec, vspec, wspec, vspec, wspec, vspec,
                  vspec, vspec],
        out_specs=pl.BlockSpec((blk, FUSED), lambda i: (i, 0)),
        out_shape=jax.ShapeDtypeStruct((Bq, FUSED), jnp.float32),
    )(x, fp['ln_g'], fp['ln_b'], fp['W1'], fp['b1'], fp['W2'], fp['b2'],
      fp['W3'], fp['b3'], fp['norm_g'], fp['norm_b'])


def _transformer(x, p):
    Bq, Sq, Cq = x.shape; dh = Cq // NHEAD
    qkv = x @ p['Wqkv'].T + p['bqkv']
    q, k, v = jnp.split(qkv, 3, axis=-1)
    rs = lambda t: t.reshape(Bq, Sq, NHEAD, dh).transpose(0, 2, 1, 3)
    q, k, v = rs(q), rs(k), rs(v)
    a = jax.nn.softmax(q @ k.transpose(0, 1, 3, 2) / jnp.sqrt(float(dh)), axis=-1)
    o = (a @ v).transpose(0, 2, 1, 3).reshape(Bq, Sq, Cq)
    o = o @ p['Wo'].T + p['bo']
    x = _ln(x + o, p['ln1_g'], p['ln1_b'])
    f = jax.nn.relu(x @ p['W1'].T + p['b1']) @ p['W2'].T + p['b2']
    return _ln(x + f, p['ln2_g'], p['ln2_b'])


def _pna(x, edge_index, edge_attr, p):
    src, dst = edge_index[0], edge_index[1]
    ea = edge_attr @ p['We'].T + p['be']
    h = jnp.concatenate([x[dst], x[src], ea], axis=-1) @ p['Wpre'].T + p['bpre']
    n = x.shape[0]
    cnt = jax.ops.segment_sum(jnp.ones((h.shape[0],), h.dtype), dst, num_segments=n)
    cntc = jnp.maximum(cnt, 1.0)[:, None]
    mean = jax.ops.segment_sum(h, dst, num_segments=n) / cntc
    mx = jnp.where(cnt[:, None] > 0, jax.ops.segment_max(h, dst, num_segments=n), 0.0)
    mn = jnp.where(cnt[:, None] > 0, jax.ops.segment_min(h, dst, num_segments=n), 0.0)
    msq = jax.ops.segment_sum(h * h, dst, num_segments=n) / cntc
    std = jnp.sqrt(jax.nn.relu(msq - mean * mean) + 1e-5)
    agg = jnp.concatenate([mean, mx, mn, std], axis=-1)
    amp = agg * (jnp.log(cntc + 1.0) / AVG_LOG)
    att = agg * (AVG_LOG / jnp.log(cntc + 1.0))
    out = jnp.concatenate([x, agg, amp, att], axis=-1) @ p['Wpost'].T + p['bpost']
    return out @ p['Wlin'].T + p['blin']


def kernel(x_tab, x_gnn, edge_index, edge_attr, params):
    t = _transformer(x_tab, params['tab'])
    t = _ln(t, params['tab_norm_g'], params['tab_norm_b'])
    cls, rest = t[:, 0, :], t[:, 1:, :]
    g = _pna(x_gnn, edge_index, edge_attr, params['pna'])
    g = g / jnp.sqrt(1.0 + 1e-5) * params['bn_g'] + params['bn_b']
    g = (x_gnn + jax.nn.relu(g)) / 2.0
    b = cls.shape[0]
    src_b, dst_b = edge_index[0][:b], edge_index[1][:b]
    x = jnp.concatenate([cls, g[src_b], g[dst_b]], axis=-1)
    x = _fuse_mlp(x, params['fuse'])
    x_tab_out = jnp.concatenate([x[:, :C][:, None, :], rest], axis=1)
    g = g.at[src_b].set(x[:, C:C + H])
    g = g.at[dst_b].set(x[:, C + H:])
    return (x_tab_out, g, edge_attr)
